# baseline (device time: 108572 ns/iter reference)
import jax
import jax.numpy as jnp
from jax import lax
from jax.experimental import pallas as pl
from jax.experimental.pallas import tpu as pltpu

NZ = 4


def kernel(x, assign, W1, W2):
    T, D = x.shape
    E, _, F = W1.shape
    NE = NZ * E

    mask8 = (assign[:, None] == jnp.arange(NE, dtype=assign.dtype)[None, :])
    mask8 = mask8.astype(jnp.float32)

    def body(x_ref, m_ref, w1_ref, w2_ref, out_ref,
             xall, mall, part, rsb,
             sx, rx, sm, rm, srs, rrs):
        my_x = lax.axis_index("x")
        my_y = lax.axis_index("y")
        my_z = lax.axis_index("z")
        left = (my_z + NZ - 1) % NZ
        right = (my_z + 1) % NZ

        barrier = pltpu.get_barrier_semaphore()
        for nbr in (left, right):
            pl.semaphore_signal(
                barrier, inc=1,
                device_id=(my_x, my_y, nbr),
                device_id_type=pl.DeviceIdType.MESH,
            )
        pl.semaphore_wait(barrier, 2)

        xall[pl.ds(my_z, 1)] = x_ref[...][None]
        mall[pl.ds(my_z, 1)] = m_ref[...][None]

        for h in range(NZ - 1):
            o = (my_z + NZ - h) % NZ
            rdma_x = pltpu.make_async_remote_copy(
                src_ref=xall.at[o], dst_ref=xall.at[o],
                send_sem=sx.at[h], recv_sem=rx.at[h],
                device_id=(my_x, my_y, right),
                device_id_type=pl.DeviceIdType.MESH,
            )
            rdma_m = pltpu.make_async_remote_copy(
                src_ref=mall.at[o], dst_ref=mall.at[o],
                send_sem=sm.at[h], recv_sem=rm.at[h],
                device_id=(my_x, my_y, right),
                device_id_type=pl.DeviceIdType.MESH,
            )
            rdma_x.start()
            rdma_m.start()
            rdma_x.wait()
            rdma_m.wait()

        iota8 = lax.broadcasted_iota(jnp.int32, (T, NE), 1)
        for c in range(NZ):
            xc = xall[c]
            m8 = mall[c]
            acc = None
            for e_local in range(E):
                e_glob = my_z * E + e_local
                msel = jnp.sum(
                    jnp.where(iota8 == e_glob, m8, 0.0),
                    axis=1, keepdims=True,
                )
                xm = xc * msel
                hh = jnp.maximum(
                    jnp.dot(xm, w1_ref[e_local],
                            preferred_element_type=jnp.float32),
                    0.0,
                )
                pe = jnp.dot(hh, w2_ref[e_local],
                             preferred_element_type=jnp.float32)
                acc = pe if acc is None else acc + pe
            part[c] = acc

        rsb[pl.ds(0, 1)] = part[pl.ds((my_z + NZ - 1) % NZ, 1)]
        for s in range(NZ - 1):
            rdma = pltpu.make_async_remote_copy(
                src_ref=rsb.at[s], dst_ref=rsb.at[s + 1],
                send_sem=srs.at[s], recv_sem=rrs.at[s],
                device_id=(my_x, my_y, right),
                device_id_type=pl.DeviceIdType.MESH,
            )
            rdma.start()
            rdma.wait()
            add = part[pl.ds((my_z + 2 - s) % NZ, 1)][0]
            if s < NZ - 2:
                rsb[s + 1] = rsb[s + 1] + add
            else:
                out_ref[...] = rsb[s + 1] + add

    return pl.pallas_call(
        body,
        out_shape=jax.ShapeDtypeStruct((T, D), jnp.float32),
        in_specs=[
            pl.BlockSpec(memory_space=pltpu.VMEM),
            pl.BlockSpec(memory_space=pltpu.VMEM),
            pl.BlockSpec(memory_space=pltpu.VMEM),
            pl.BlockSpec(memory_space=pltpu.VMEM),
        ],
        out_specs=pl.BlockSpec(memory_space=pltpu.VMEM),
        scratch_shapes=[
            pltpu.VMEM((NZ, T, D), jnp.float32),
            pltpu.VMEM((NZ, T, NE), jnp.float32),
            pltpu.VMEM((NZ, T, D), jnp.float32),
            pltpu.VMEM((NZ, T, D), jnp.float32),
            pltpu.SemaphoreType.DMA((NZ - 1,)),
            pltpu.SemaphoreType.DMA((NZ - 1,)),
            pltpu.SemaphoreType.DMA((NZ - 1,)),
            pltpu.SemaphoreType.DMA((NZ - 1,)),
            pltpu.SemaphoreType.DMA((NZ - 1,)),
            pltpu.SemaphoreType.DMA((NZ - 1,)),
        ],
        compiler_params=pltpu.CompilerParams(collective_id=0),
    )(x, mask8, W1, W2)


# device time: 55165 ns/iter; 1.9681x vs baseline; 1.9681x over previous
import jax
import jax.numpy as jnp
from jax import lax
from jax.experimental import pallas as pl
from jax.experimental.pallas import tpu as pltpu

NZ = 4


def kernel(x, assign, W1, W2):
    T, D = x.shape
    E, _, F = W1.shape
    NE = NZ * E

    mask8 = (assign[:, None] == jnp.arange(NE, dtype=assign.dtype)[None, :])
    mask8 = mask8.astype(jnp.bfloat16)

    def body(x_ref, m_ref, w1_ref, w2_ref, out_ref,
             xall, mall, part, rsb, w1b, w2b,
             sx, rx, sm, rm, srs, rrs):
        my_x = lax.axis_index("x")
        my_y = lax.axis_index("y")
        my_z = lax.axis_index("z")
        left = (my_z + NZ - 1) % NZ
        right = (my_z + 1) % NZ

        barrier = pltpu.get_barrier_semaphore()
        for nbr in (left, right):
            pl.semaphore_signal(
                barrier, inc=1,
                device_id=(my_x, my_y, nbr),
                device_id_type=pl.DeviceIdType.MESH,
            )
        pl.semaphore_wait(barrier, 2)

        xall[pl.ds(my_z, 1)] = x_ref[...].astype(jnp.bfloat16)[None]
        mall[pl.ds(my_z, 1)] = m_ref[...][None]
        w1b[...] = w1_ref[...].astype(jnp.bfloat16)
        w2b[...] = w2_ref[...].astype(jnp.bfloat16)

        iota8 = lax.broadcasted_iota(jnp.int32, (T, NE), 1)

        def compute_chunk(c):
            xc = xall[pl.ds(c, 1)][0]
            m8 = mall[pl.ds(c, 1)][0]
            acc = None
            for e_local in range(E):
                e_glob = my_z * E + e_local
                msel = jnp.sum(
                    jnp.where(iota8 == e_glob, m8, jnp.bfloat16(0)),
                    axis=1, keepdims=True,
                )
                xm = xc * msel
                hh = jnp.maximum(
                    jnp.dot(xm, w1b[e_local],
                            preferred_element_type=jnp.float32),
                    0.0,
                ).astype(jnp.bfloat16)
                pe = jnp.dot(hh, w2b[e_local],
                             preferred_element_type=jnp.float32)
                acc = pe if acc is None else acc + pe
            part[pl.ds(c, 1)] = acc[None]

        def ag_hop(h):
            o = (my_z + NZ - h) % NZ
            rdma_x = pltpu.make_async_remote_copy(
                src_ref=xall.at[o], dst_ref=xall.at[o],
                send_sem=sx.at[h], recv_sem=rx.at[h],
                device_id=(my_x, my_y, right),
                device_id_type=pl.DeviceIdType.MESH,
            )
            rdma_m = pltpu.make_async_remote_copy(
                src_ref=mall.at[o], dst_ref=mall.at[o],
                send_sem=sm.at[h], recv_sem=rm.at[h],
                device_id=(my_x, my_y, right),
                device_id_type=pl.DeviceIdType.MESH,
            )
            rdma_x.start()
            rdma_m.start()
            return rdma_x, rdma_m

        def rs_step(s):
            rdma = pltpu.make_async_remote_copy(
                src_ref=rsb.at[s], dst_ref=rsb.at[s + 1],
                send_sem=srs.at[s], recv_sem=rrs.at[s],
                device_id=(my_x, my_y, right),
                device_id_type=pl.DeviceIdType.MESH,
            )
            rdma.start()
            return rdma

        ag0x, ag0m = ag_hop(0)
        compute_chunk(my_z)
        ag0x.wait()
        ag0m.wait()

        ag1x, ag1m = ag_hop(1)
        compute_chunk((my_z + NZ - 1) % NZ)
        rsb[pl.ds(0, 1)] = part[pl.ds((my_z + NZ - 1) % NZ, 1)].astype(
            jnp.bfloat16)
        rs0 = rs_step(0)
        ag1x.wait()
        ag1m.wait()

        ag2x, ag2m = ag_hop(2)
        compute_chunk((my_z + NZ - 2) % NZ)
        rs0.wait()
        rsb[pl.ds(1, 1)] = (
            rsb[pl.ds(1, 1)].astype(jnp.float32)
            + part[pl.ds((my_z + NZ - 2) % NZ, 1)]
        ).astype(jnp.bfloat16)
        rs1 = rs_step(1)
        ag2x.wait()
        ag2m.wait()

        compute_chunk((my_z + 1) % NZ)
        rs1.wait()
        rsb[pl.ds(2, 1)] = (
            rsb[pl.ds(2, 1)].astype(jnp.float32)
            + part[pl.ds((my_z + 1) % NZ, 1)]
        ).astype(jnp.bfloat16)
        rs2 = rs_step(2)
        rs2.wait()
        out_ref[...] = (
            rsb[pl.ds(3, 1)][0].astype(jnp.float32)
            + part[pl.ds(my_z, 1)][0]
        )

    return pl.pallas_call(
        body,
        out_shape=jax.ShapeDtypeStruct((T, D), jnp.float32),
        in_specs=[
            pl.BlockSpec(memory_space=pltpu.VMEM),
            pl.BlockSpec(memory_space=pltpu.VMEM),
            pl.BlockSpec(memory_space=pltpu.VMEM),
            pl.BlockSpec(memory_space=pltpu.VMEM),
        ],
        out_specs=pl.BlockSpec(memory_space=pltpu.VMEM),
        scratch_shapes=[
            pltpu.VMEM((NZ, T, D), jnp.bfloat16),
            pltpu.VMEM((NZ, T, NE), jnp.bfloat16),
            pltpu.VMEM((NZ, T, D), jnp.float32),
            pltpu.VMEM((NZ, T, D), jnp.bfloat16),
            pltpu.VMEM((E, D, F), jnp.bfloat16),
            pltpu.VMEM((E, F, D), jnp.bfloat16),
            pltpu.SemaphoreType.DMA((NZ - 1,)),
            pltpu.SemaphoreType.DMA((NZ - 1,)),
            pltpu.SemaphoreType.DMA((NZ - 1,)),
            pltpu.SemaphoreType.DMA((NZ - 1,)),
            pltpu.SemaphoreType.DMA((NZ - 1,)),
            pltpu.SemaphoreType.DMA((NZ - 1,)),
        ],
        compiler_params=pltpu.CompilerParams(collective_id=0),
    )(x, mask8, W1, W2)


# device time: 53966 ns/iter; 2.0119x vs baseline; 1.0222x over previous
import jax
import jax.numpy as jnp
from jax import lax
from jax.experimental import pallas as pl
from jax.experimental.pallas import tpu as pltpu

NZ = 4
NH = 2


def kernel(x, assign, W1, W2):
    T, D = x.shape
    E, _, F = W1.shape
    NE = NZ * E
    TH = T // NH

    mask8 = (assign[:, None] == jnp.arange(NE, dtype=assign.dtype)[None, :])
    mask8 = mask8.astype(jnp.bfloat16)

    def body(x_ref, m_ref, w1_ref, w2_ref, out_ref,
             xall, mall, part, rsb, w1b, w2b,
             sx, rx, sm, rm, srs, rrs):
        my_x = lax.axis_index("x")
        my_y = lax.axis_index("y")
        my_z = lax.axis_index("z")
        left = (my_z + NZ - 1) % NZ
        right = (my_z + 1) % NZ
        zm1 = (my_z + NZ - 1) % NZ
        zm2 = (my_z + NZ - 2) % NZ
        zp1 = (my_z + 1) % NZ

        barrier = pltpu.get_barrier_semaphore()
        for nbr in (left, right):
            pl.semaphore_signal(
                barrier, inc=1,
                device_id=(my_x, my_y, nbr),
                device_id_type=pl.DeviceIdType.MESH,
            )
        pl.semaphore_wait(barrier, 2)

        xall[pl.ds(my_z, 1)] = x_ref[...].astype(jnp.bfloat16)[None]
        mall[pl.ds(my_z, 1)] = m_ref[...][None]
        w1b[...] = w1_ref[...].astype(jnp.bfloat16)
        w2b[...] = w2_ref[...].astype(jnp.bfloat16)

        iota8 = lax.broadcasted_iota(jnp.int32, (TH, NE), 1)

        def compute_half(c, q):
            xc = xall[pl.ds(c, 1), pl.ds(q * TH, TH)][0]
            m8 = mall[pl.ds(c, 1), pl.ds(q * TH, TH)][0]
            acc = None
            for e_local in range(E):
                e_glob = my_z * E + e_local
                msel = jnp.sum(
                    jnp.where(iota8 == e_glob, m8, jnp.bfloat16(0)),
                    axis=1, keepdims=True,
                )
                xm = xc * msel
                hh = jnp.maximum(
                    jnp.dot(xm, w1b[e_local],
                            preferred_element_type=jnp.float32),
                    0.0,
                ).astype(jnp.bfloat16)
                pe = jnp.dot(hh, w2b[e_local],
                             preferred_element_type=jnp.float32)
                acc = pe if acc is None else acc + pe
            part[pl.ds(c, 1), pl.ds(q * TH, TH)] = acc[None]

        def ag_x(h, q):
            o = (my_z + NZ - h) % NZ
            rdma = pltpu.make_async_remote_copy(
                src_ref=xall.at[o, pl.ds(q * TH, TH)],
                dst_ref=xall.at[o, pl.ds(q * TH, TH)],
                send_sem=sx.at[h, q], recv_sem=rx.at[h, q],
                device_id=(my_x, my_y, right),
                device_id_type=pl.DeviceIdType.MESH,
            )
            rdma.start()
            return rdma

        def ag_m(h):
            o = (my_z + NZ - h) % NZ
            rdma = pltpu.make_async_remote_copy(
                src_ref=mall.at[o], dst_ref=mall.at[o],
                send_sem=sm.at[h], recv_sem=rm.at[h],
                device_id=(my_x, my_y, right),
                device_id_type=pl.DeviceIdType.MESH,
            )
            rdma.start()
            return rdma

        def rs_step(s, q):
            rdma = pltpu.make_async_remote_copy(
                src_ref=rsb.at[s, pl.ds(q * TH, TH)],
                dst_ref=rsb.at[s + 1, pl.ds(q * TH, TH)],
                send_sem=srs.at[s, q], recv_sem=rrs.at[s, q],
                device_id=(my_x, my_y, right),
                device_id_type=pl.DeviceIdType.MESH,
            )
            rdma.start()
            return rdma

        def rs_stage0(q):
            rsb[pl.ds(0, 1), pl.ds(q * TH, TH)] = part[
                pl.ds(zm1, 1), pl.ds(q * TH, TH)].astype(jnp.bfloat16)
            return rs_step(0, q)

        def rs_acc(s, q, c):
            rsb[pl.ds(s + 1, 1), pl.ds(q * TH, TH)] = (
                rsb[pl.ds(s + 1, 1), pl.ds(q * TH, TH)].astype(jnp.float32)
                + part[pl.ds(c, 1), pl.ds(q * TH, TH)]
            ).astype(jnp.bfloat16)
            return rs_step(s + 1, q)

        a0h0 = ag_x(0, 0)
        a0h1 = ag_x(0, 1)
        m0 = ag_m(0)
        compute_half(my_z, 0)
        compute_half(my_z, 1)

        a0h0.wait()
        m0.wait()
        m1 = ag_m(1)
        a1h0 = ag_x(1, 0)
        compute_half(zm1, 0)
        r0h0 = rs_stage0(0)

        a0h1.wait()
        a1h1 = ag_x(1, 1)
        compute_half(zm1, 1)
        r0h1 = rs_stage0(1)

        a1h0.wait()
        m1.wait()
        m2 = ag_m(2)
        a2h0 = ag_x(2, 0)
        compute_half(zm2, 0)
        r0h0.wait()
        r1h0 = rs_acc(0, 0, zm2)

        a1h1.wait()
        a2h1 = ag_x(2, 1)
        compute_half(zm2, 1)
        r0h1.wait()
        r1h1 = rs_acc(0, 1, zm2)

        a2h0.wait()
        m2.wait()
        compute_half(zp1, 0)
        r1h0.wait()
        r2h0 = rs_acc(1, 0, zp1)

        a2h1.wait()
        compute_half(zp1, 1)
        r1h1.wait()
        r2h1 = rs_acc(1, 1, zp1)

        r2h0.wait()
        out_ref[pl.ds(0, TH)] = (
            rsb[pl.ds(3, 1), pl.ds(0, TH)][0].astype(jnp.float32)
            + part[pl.ds(my_z, 1), pl.ds(0, TH)][0]
        )
        r2h1.wait()
        out_ref[pl.ds(TH, TH)] = (
            rsb[pl.ds(3, 1), pl.ds(TH, TH)][0].astype(jnp.float32)
            + part[pl.ds(my_z, 1), pl.ds(TH, TH)][0]
        )

    return pl.pallas_call(
        body,
        out_shape=jax.ShapeDtypeStruct((T, D), jnp.float32),
        in_specs=[
            pl.BlockSpec(memory_space=pltpu.VMEM),
            pl.BlockSpec(memory_space=pltpu.VMEM),
            pl.BlockSpec(memory_space=pltpu.VMEM),
            pl.BlockSpec(memory_space=pltpu.VMEM),
        ],
        out_specs=pl.BlockSpec(memory_space=pltpu.VMEM),
        scratch_shapes=[
            pltpu.VMEM((NZ, T, D), jnp.bfloat16),
            pltpu.VMEM((NZ, T, NE), jnp.bfloat16),
            pltpu.VMEM((NZ, T, D), jnp.float32),
            pltpu.VMEM((NZ, T, D), jnp.bfloat16),
            pltpu.VMEM((E, D, F), jnp.bfloat16),
            pltpu.VMEM((E, F, D), jnp.bfloat16),
            pltpu.SemaphoreType.DMA((NZ - 1, NH)),
            pltpu.SemaphoreType.DMA((NZ - 1, NH)),
            pltpu.SemaphoreType.DMA((NZ - 1,)),
            pltpu.SemaphoreType.DMA((NZ - 1,)),
            pltpu.SemaphoreType.DMA((NZ - 1, NH)),
            pltpu.SemaphoreType.DMA((NZ - 1, NH)),
        ],
        compiler_params=pltpu.CompilerParams(collective_id=0),
    )(x, mask8, W1, W2)


# device time: 34164 ns/iter; 3.1780x vs baseline; 1.5796x over previous
import jax
import jax.numpy as jnp
from jax import lax
from jax.experimental import pallas as pl
from jax.experimental.pallas import tpu as pltpu

NZ = 4
NQ = 4


def kernel(x, assign, W1, W2):
    T, D = x.shape
    E, _, F = W1.shape
    NE = NZ * E
    TQ = T // NQ

    mask8 = (assign[:, None] == jnp.arange(NE, dtype=assign.dtype)[None, :])
    mask8 = mask8.astype(jnp.bfloat16)

    def body(x_ref, m_ref, w1_ref, w2_ref, out_ref,
             xall, mall, part, rsstage, rbuf, gbuf, w1b, w2b,
             sxd, rxd, smd, rmd, srd, rrd, sg, rg):
        my_x = lax.axis_index("x")
        my_y = lax.axis_index("y")
        my_z = lax.axis_index("z")
        qid = my_y * 2 + my_x

        barrier = pltpu.get_barrier_semaphore()
        for d in range(1, NZ):
            pl.semaphore_signal(
                barrier, inc=1,
                device_id=(my_x, my_y, (my_z + d) % NZ),
                device_id_type=pl.DeviceIdType.MESH,
            )
        pl.semaphore_signal(
            barrier, inc=1, device_id=(1 - my_x, my_y, my_z),
            device_id_type=pl.DeviceIdType.MESH,
        )
        pl.semaphore_signal(
            barrier, inc=1, device_id=(my_x, 1 - my_y, my_z),
            device_id_type=pl.DeviceIdType.MESH,
        )
        pl.semaphore_wait(barrier, NZ + 1)

        xall[pl.ds(my_z, 1)] = x_ref[pl.ds(qid * TQ, TQ)].astype(
            jnp.bfloat16)[None]
        mall[pl.ds(my_z, 1)] = m_ref[pl.ds(qid * TQ, TQ)][None]

        def ag_send(d):
            tgt = (my_z + d) % NZ
            rdma_x = pltpu.make_async_remote_copy(
                src_ref=xall.at[my_z], dst_ref=xall.at[my_z],
                send_sem=sxd.at[d - 1], recv_sem=rxd.at[d - 1],
                device_id=(my_x, my_y, tgt),
                device_id_type=pl.DeviceIdType.MESH,
            )
            rdma_m = pltpu.make_async_remote_copy(
                src_ref=mall.at[my_z], dst_ref=mall.at[my_z],
                send_sem=smd.at[d - 1], recv_sem=rmd.at[d - 1],
                device_id=(my_x, my_y, tgt),
                device_id_type=pl.DeviceIdType.MESH,
            )
            rdma_x.start()
            rdma_m.start()
            return rdma_x, rdma_m

        ag = {d: ag_send(d) for d in (1, 2, 3)}

        w1b[...] = w1_ref[...].astype(jnp.bfloat16)
        w2b[...] = w2_ref[...].astype(jnp.bfloat16)
        iota8 = lax.broadcasted_iota(jnp.int32, (TQ, NE), 1)

        def compute_chunk(c):
            xc = xall[pl.ds(c, 1)][0]
            m8 = mall[pl.ds(c, 1)][0]
            acc = None
            for e_local in range(E):
                e_glob = my_z * E + e_local
                msel = jnp.sum(
                    jnp.where(iota8 == e_glob, m8, jnp.bfloat16(0)),
                    axis=1, keepdims=True,
                )
                xm = xc * msel
                hh = jnp.maximum(
                    jnp.dot(xm, w1b[e_local],
                            preferred_element_type=jnp.float32),
                    0.0,
                ).astype(jnp.bfloat16)
                pe = jnp.dot(hh, w2b[e_local],
                             preferred_element_type=jnp.float32)
                acc = pe if acc is None else acc + pe
            part[pl.ds(c, 1)] = acc[None]

        def rs_send(d):
            c = (my_z + d) % NZ
            rsstage[pl.ds(d - 1, 1)] = part[pl.ds(c, 1)].astype(jnp.bfloat16)
            rdma = pltpu.make_async_remote_copy(
                src_ref=rsstage.at[d - 1], dst_ref=rbuf.at[d - 1],
                send_sem=srd.at[d - 1], recv_sem=rrd.at[d - 1],
                device_id=(my_x, my_y, c),
                device_id_type=pl.DeviceIdType.MESH,
            )
            rdma.start()
            return rdma

        compute_chunk(my_z)

        rs = {}
        ag[1][0].wait()
        ag[1][1].wait()
        compute_chunk((my_z + NZ - 1) % NZ)
        rs[3] = rs_send(3)
        ag[3][0].wait()
        ag[3][1].wait()
        compute_chunk((my_z + 1) % NZ)
        rs[1] = rs_send(1)
        ag[2][0].wait()
        ag[2][1].wait()
        compute_chunk((my_z + 2) % NZ)
        rs[2] = rs_send(2)

        for d in (1, 2, 3):
            rs[d].wait()
        res = (
            part[pl.ds(my_z, 1)][0]
            + rbuf[0].astype(jnp.float32)
            + rbuf[1].astype(jnp.float32)
            + rbuf[2].astype(jnp.float32)
        )
        gbuf[pl.ds(qid * TQ, TQ)] = res.astype(jnp.bfloat16)

        gx = pltpu.make_async_remote_copy(
            src_ref=gbuf.at[pl.ds(qid * TQ, TQ)],
            dst_ref=gbuf.at[pl.ds(qid * TQ, TQ)],
            send_sem=sg.at[0], recv_sem=rg.at[0],
            device_id=(1 - my_x, my_y, my_z),
            device_id_type=pl.DeviceIdType.MESH,
        )
        gx.start()
        gx.wait()
        gy = pltpu.make_async_remote_copy(
            src_ref=gbuf.at[pl.ds(my_y * 2 * TQ, 2 * TQ)],
            dst_ref=gbuf.at[pl.ds(my_y * 2 * TQ, 2 * TQ)],
            send_sem=sg.at[1], recv_sem=rg.at[1],
            device_id=(my_x, 1 - my_y, my_z),
            device_id_type=pl.DeviceIdType.MESH,
        )
        gy.start()
        gy.wait()
        out_ref[...] = gbuf[...].astype(jnp.float32)

    return pl.pallas_call(
        body,
        out_shape=jax.ShapeDtypeStruct((T, D), jnp.float32),
        in_specs=[
            pl.BlockSpec(memory_space=pltpu.VMEM),
            pl.BlockSpec(memory_space=pltpu.VMEM),
            pl.BlockSpec(memory_space=pltpu.VMEM),
            pl.BlockSpec(memory_space=pltpu.VMEM),
        ],
        out_specs=pl.BlockSpec(memory_space=pltpu.VMEM),
        scratch_shapes=[
            pltpu.VMEM((NZ, TQ, D), jnp.bfloat16),
            pltpu.VMEM((NZ, TQ, NE), jnp.bfloat16),
            pltpu.VMEM((NZ, TQ, D), jnp.float32),
            pltpu.VMEM((NZ - 1, TQ, D), jnp.bfloat16),
            pltpu.VMEM((NZ - 1, TQ, D), jnp.bfloat16),
            pltpu.VMEM((T, D), jnp.bfloat16),
            pltpu.VMEM((E, D, F), jnp.bfloat16),
            pltpu.VMEM((E, F, D), jnp.bfloat16),
            pltpu.SemaphoreType.DMA((NZ - 1,)),
            pltpu.SemaphoreType.DMA((NZ - 1,)),
            pltpu.SemaphoreType.DMA((NZ - 1,)),
            pltpu.SemaphoreType.DMA((NZ - 1,)),
            pltpu.SemaphoreType.DMA((NZ - 1,)),
            pltpu.SemaphoreType.DMA((NZ - 1,)),
            pltpu.SemaphoreType.DMA((2,)),
            pltpu.SemaphoreType.DMA((2,)),
        ],
        compiler_params=pltpu.CompilerParams(collective_id=0),
    )(x, mask8, W1, W2)


# device time: 32303 ns/iter; 3.3611x vs baseline; 1.0576x over previous
import jax
import jax.numpy as jnp
from jax import lax
from jax.experimental import pallas as pl
from jax.experimental.pallas import tpu as pltpu

NZ = 4
NQ = 4


def kernel(x, assign, W1, W2):
    T, D = x.shape
    E, _, F = W1.shape
    NE = NZ * E
    TQ = T // NQ

    mask8 = (assign[:, None] == jnp.arange(NE, dtype=assign.dtype)[None, :])
    mask8 = mask8.astype(jnp.bfloat16)

    def body(x_ref, m_ref, w1_ref, w2_ref, out_ref,
             xall, mall, part, rsstage, rbuf, gbuf, w1f, w2f, w1b, w2b,
             sxd, rxd, smd, rmd, srd, rrd, sg, rg, swl):
        my_x = lax.axis_index("x")
        my_y = lax.axis_index("y")
        my_z = lax.axis_index("z")
        qid = my_y * 2 + my_x

        w1cp = pltpu.make_async_copy(w1_ref, w1f, swl.at[0])
        w2cp = pltpu.make_async_copy(w2_ref, w2f, swl.at[1])
        w1cp.start()
        w2cp.start()

        barrier = pltpu.get_barrier_semaphore()
        for d in range(1, NZ):
            pl.semaphore_signal(
                barrier, inc=1,
                device_id=(my_x, my_y, (my_z + d) % NZ),
                device_id_type=pl.DeviceIdType.MESH,
            )
        for dev in ((1 - my_x, my_y, my_z), (my_x, 1 - my_y, my_z),
                    (1 - my_x, 1 - my_y, my_z)):
            pl.semaphore_signal(
                barrier, inc=1, device_id=dev,
                device_id_type=pl.DeviceIdType.MESH,
            )
        pl.semaphore_wait(barrier, NZ + 2)

        xall[pl.ds(my_z, 1)] = x_ref[pl.ds(qid * TQ, TQ)].astype(
            jnp.bfloat16)[None]
        mall[pl.ds(my_z, 1)] = m_ref[pl.ds(qid * TQ, TQ)][None]

        def ag_send(d):
            tgt = (my_z + d) % NZ
            rdma_x = pltpu.make_async_remote_copy(
                src_ref=xall.at[my_z], dst_ref=xall.at[my_z],
                send_sem=sxd.at[d - 1], recv_sem=rxd.at[d - 1],
                device_id=(my_x, my_y, tgt),
                device_id_type=pl.DeviceIdType.MESH,
            )
            rdma_m = pltpu.make_async_remote_copy(
                src_ref=mall.at[my_z], dst_ref=mall.at[my_z],
                send_sem=smd.at[d - 1], recv_sem=rmd.at[d - 1],
                device_id=(my_x, my_y, tgt),
                device_id_type=pl.DeviceIdType.MESH,
            )
            rdma_x.start()
            rdma_m.start()
            return rdma_x, rdma_m

        ag = {d: ag_send(d) for d in (1, 2, 3)}

        w1cp.wait()
        w2cp.wait()
        w1b[...] = w1f[...].astype(jnp.bfloat16)
        w2b[...] = w2f[...].astype(jnp.bfloat16)
        iota8 = lax.broadcasted_iota(jnp.int32, (TQ, NE), 1)

        def compute_chunk(c):
            xc = xall[pl.ds(c, 1)][0]
            m8 = mall[pl.ds(c, 1)][0]
            acc = None
            for e_local in range(E):
                e_glob = my_z * E + e_local
                msel = jnp.sum(
                    jnp.where(iota8 == e_glob, m8, jnp.bfloat16(0)),
                    axis=1, keepdims=True,
                )
                xm = xc * msel
                hh = jnp.maximum(
                    jnp.dot(xm, w1b[e_local],
                            preferred_element_type=jnp.float32),
                    0.0,
                ).astype(jnp.bfloat16)
                pe = jnp.dot(hh, w2b[e_local],
                             preferred_element_type=jnp.float32)
                acc = pe if acc is None else acc + pe
            part[pl.ds(c, 1)] = acc[None]

        def rs_send(d):
            c = (my_z + d) % NZ
            rsstage[pl.ds(d - 1, 1)] = part[pl.ds(c, 1)].astype(jnp.bfloat16)
            rdma = pltpu.make_async_remote_copy(
                src_ref=rsstage.at[d - 1], dst_ref=rbuf.at[d - 1],
                send_sem=srd.at[d - 1], recv_sem=rrd.at[d - 1],
                device_id=(my_x, my_y, c),
                device_id_type=pl.DeviceIdType.MESH,
            )
            rdma.start()
            return rdma

        compute_chunk(my_z)

        rs = {}
        ag[1][0].wait()
        ag[1][1].wait()
        compute_chunk((my_z + NZ - 1) % NZ)
        rs[3] = rs_send(3)
        ag[3][0].wait()
        ag[3][1].wait()
        compute_chunk((my_z + 1) % NZ)
        rs[1] = rs_send(1)
        ag[2][0].wait()
        ag[2][1].wait()
        compute_chunk((my_z + 2) % NZ)
        rs[2] = rs_send(2)

        for d in (1, 2, 3):
            rs[d].wait()
        res = (
            part[pl.ds(my_z, 1)][0]
            + rbuf[0].astype(jnp.float32)
            + rbuf[1].astype(jnp.float32)
            + rbuf[2].astype(jnp.float32)
        )
        gbuf[pl.ds(qid * TQ, TQ)] = res.astype(jnp.bfloat16)

        peers = ((1 - my_x, my_y, my_z), (my_x, 1 - my_y, my_z),
                 (1 - my_x, 1 - my_y, my_z))
        gs = []
        for i, dev in enumerate(peers):
            g = pltpu.make_async_remote_copy(
                src_ref=gbuf.at[pl.ds(qid * TQ, TQ)],
                dst_ref=gbuf.at[pl.ds(qid * TQ, TQ)],
                send_sem=sg.at[i], recv_sem=rg.at[i],
                device_id=dev,
                device_id_type=pl.DeviceIdType.MESH,
            )
            g.start()
            gs.append(g)
        for g in gs:
            g.wait()
        out_ref[...] = gbuf[...].astype(jnp.float32)

    return pl.pallas_call(
        body,
        out_shape=jax.ShapeDtypeStruct((T, D), jnp.float32),
        in_specs=[
            pl.BlockSpec(memory_space=pltpu.VMEM),
            pl.BlockSpec(memory_space=pltpu.VMEM),
            pl.BlockSpec(memory_space=pl.ANY),
            pl.BlockSpec(memory_space=pl.ANY),
        ],
        out_specs=pl.BlockSpec(memory_space=pltpu.VMEM),
        scratch_shapes=[
            pltpu.VMEM((NZ, TQ, D), jnp.bfloat16),
            pltpu.VMEM((NZ, TQ, NE), jnp.bfloat16),
            pltpu.VMEM((NZ, TQ, D), jnp.float32),
            pltpu.VMEM((NZ - 1, TQ, D), jnp.bfloat16),
            pltpu.VMEM((NZ - 1, TQ, D), jnp.bfloat16),
            pltpu.VMEM((T, D), jnp.bfloat16),
            pltpu.VMEM((E, D, F), jnp.float32),
            pltpu.VMEM((E, F, D), jnp.float32),
            pltpu.VMEM((E, D, F), jnp.bfloat16),
            pltpu.VMEM((E, F, D), jnp.bfloat16),
            pltpu.SemaphoreType.DMA((NZ - 1,)),
            pltpu.SemaphoreType.DMA((NZ - 1,)),
            pltpu.SemaphoreType.DMA((NZ - 1,)),
            pltpu.SemaphoreType.DMA((NZ - 1,)),
            pltpu.SemaphoreType.DMA((NZ - 1,)),
            pltpu.SemaphoreType.DMA((NZ - 1,)),
            pltpu.SemaphoreType.DMA((3,)),
            pltpu.SemaphoreType.DMA((3,)),
            pltpu.SemaphoreType.DMA((2,)),
        ],
        compiler_params=pltpu.CompilerParams(collective_id=0),
    )(x, mask8, W1, W2)


# device time: 29276 ns/iter; 3.7086x vs baseline; 1.1034x over previous
import jax
import jax.numpy as jnp
from jax import lax
from jax.experimental import pallas as pl
from jax.experimental.pallas import tpu as pltpu

NZ = 4
NQ = 4


def kernel(x, assign, W1, W2):
    T, D = x.shape
    E, _, F = W1.shape
    NE = NZ * E
    TQ = T // NQ

    qid_out = lax.axis_index("y") * 2 + lax.axis_index("x")
    xq = lax.dynamic_slice(x, (qid_out * TQ, 0), (TQ, D))
    aq = lax.dynamic_slice(assign, (qid_out * TQ,), (TQ,))
    xb = xq.astype(jnp.bfloat16)
    w1b = W1.astype(jnp.bfloat16)
    w2b = W2.astype(jnp.bfloat16)
    mask8 = (aq[:, None] == jnp.arange(NE, dtype=assign.dtype)[None, :])
    mask8 = mask8.astype(jnp.bfloat16)

    def body(x_ref, m_ref, w1_ref, w2_ref, out_ref,
             xall, mall, part, rsstage, rbuf, gbuf,
             sxd, rxd, smd, rmd, srd, rrd, sg, rg):
        my_x = lax.axis_index("x")
        my_y = lax.axis_index("y")
        my_z = lax.axis_index("z")
        qid = my_y * 2 + my_x

        barrier = pltpu.get_barrier_semaphore()
        for d in range(1, NZ):
            pl.semaphore_signal(
                barrier, inc=1,
                device_id=(my_x, my_y, (my_z + d) % NZ),
                device_id_type=pl.DeviceIdType.MESH,
            )
        for dev in ((1 - my_x, my_y, my_z), (my_x, 1 - my_y, my_z),
                    (1 - my_x, 1 - my_y, my_z)):
            pl.semaphore_signal(
                barrier, inc=1, device_id=dev,
                device_id_type=pl.DeviceIdType.MESH,
            )
        pl.semaphore_wait(barrier, NZ + 2)

        xall[pl.ds(my_z, 1)] = x_ref[...][None]
        mall[pl.ds(my_z, 1)] = m_ref[...][None]

        def ag_send(d):
            tgt = (my_z + d) % NZ
            rdma_x = pltpu.make_async_remote_copy(
                src_ref=xall.at[my_z], dst_ref=xall.at[my_z],
                send_sem=sxd.at[d - 1], recv_sem=rxd.at[d - 1],
                device_id=(my_x, my_y, tgt),
                device_id_type=pl.DeviceIdType.MESH,
            )
            rdma_m = pltpu.make_async_remote_copy(
                src_ref=mall.at[my_z], dst_ref=mall.at[my_z],
                send_sem=smd.at[d - 1], recv_sem=rmd.at[d - 1],
                device_id=(my_x, my_y, tgt),
                device_id_type=pl.DeviceIdType.MESH,
            )
            rdma_x.start()
            rdma_m.start()
            return rdma_x, rdma_m

        ag = {d: ag_send(d) for d in (1, 2, 3)}

        iota8 = lax.broadcasted_iota(jnp.int32, (TQ, NE), 1)

        def compute_chunk(c):
            xc = xall[pl.ds(c, 1)][0]
            m8 = mall[pl.ds(c, 1)][0]
            acc = None
            for e_local in range(E):
                e_glob = my_z * E + e_local
                msel = jnp.sum(
                    jnp.where(iota8 == e_glob, m8, jnp.bfloat16(0)),
                    axis=1, keepdims=True,
                )
                xm = xc * msel
                hh = jnp.maximum(
                    jnp.dot(xm, w1_ref[e_local],
                            preferred_element_type=jnp.float32),
                    0.0,
                ).astype(jnp.bfloat16)
                pe = jnp.dot(hh, w2_ref[e_local],
                             preferred_element_type=jnp.float32)
                acc = pe if acc is None else acc + pe
            part[pl.ds(c, 1)] = acc[None]

        def rs_send(d):
            c = (my_z + d) % NZ
            rsstage[pl.ds(d - 1, 1)] = part[pl.ds(c, 1)].astype(jnp.bfloat16)
            rdma = pltpu.make_async_remote_copy(
                src_ref=rsstage.at[d - 1], dst_ref=rbuf.at[d - 1],
                send_sem=srd.at[d - 1], recv_sem=rrd.at[d - 1],
                device_id=(my_x, my_y, c),
                device_id_type=pl.DeviceIdType.MESH,
            )
            rdma.start()
            return rdma

        compute_chunk(my_z)

        rs = {}
        ag[1][0].wait()
        ag[1][1].wait()
        compute_chunk((my_z + NZ - 1) % NZ)
        rs[3] = rs_send(3)
        ag[3][0].wait()
        ag[3][1].wait()
        compute_chunk((my_z + 1) % NZ)
        rs[1] = rs_send(1)
        ag[2][0].wait()
        ag[2][1].wait()
        compute_chunk((my_z + 2) % NZ)
        rs[2] = rs_send(2)

        for d in (1, 2, 3):
            rs[d].wait()
        res = (
            part[pl.ds(my_z, 1)][0]
            + rbuf[0].astype(jnp.float32)
            + rbuf[1].astype(jnp.float32)
            + rbuf[2].astype(jnp.float32)
        )
        gbuf[pl.ds(qid * TQ, TQ)] = res.astype(jnp.bfloat16)

        peers = ((1 - my_x, my_y, my_z), (my_x, 1 - my_y, my_z),
                 (1 - my_x, 1 - my_y, my_z))
        gs = []
        for i, dev in enumerate(peers):
            g = pltpu.make_async_remote_copy(
                src_ref=gbuf.at[pl.ds(qid * TQ, TQ)],
                dst_ref=gbuf.at[pl.ds(qid * TQ, TQ)],
                send_sem=sg.at[i], recv_sem=rg.at[i],
                device_id=dev,
                device_id_type=pl.DeviceIdType.MESH,
            )
            g.start()
            gs.append(g)
        for g in gs:
            g.wait()
        out_ref[...] = gbuf[...].astype(jnp.float32)

    return pl.pallas_call(
        body,
        out_shape=jax.ShapeDtypeStruct((T, D), jnp.float32),
        in_specs=[
            pl.BlockSpec(memory_space=pltpu.VMEM),
            pl.BlockSpec(memory_space=pltpu.VMEM),
            pl.BlockSpec(memory_space=pltpu.VMEM),
            pl.BlockSpec(memory_space=pltpu.VMEM),
        ],
        out_specs=pl.BlockSpec(memory_space=pltpu.VMEM),
        scratch_shapes=[
            pltpu.VMEM((NZ, TQ, D), jnp.bfloat16),
            pltpu.VMEM((NZ, TQ, NE), jnp.bfloat16),
            pltpu.VMEM((NZ, TQ, D), jnp.float32),
            pltpu.VMEM((NZ - 1, TQ, D), jnp.bfloat16),
            pltpu.VMEM((NZ - 1, TQ, D), jnp.bfloat16),
            pltpu.VMEM((T, D), jnp.bfloat16),
            pltpu.SemaphoreType.DMA((NZ - 1,)),
            pltpu.SemaphoreType.DMA((NZ - 1,)),
            pltpu.SemaphoreType.DMA((NZ - 1,)),
            pltpu.SemaphoreType.DMA((NZ - 1,)),
            pltpu.SemaphoreType.DMA((NZ - 1,)),
            pltpu.SemaphoreType.DMA((NZ - 1,)),
            pltpu.SemaphoreType.DMA((3,)),
            pltpu.SemaphoreType.DMA((3,)),
        ],
        compiler_params=pltpu.CompilerParams(collective_id=0),
    )(xb, mask8, w1b, w2b)
